# Initial kernel scaffold; baseline (speedup 1.0000x reference)
import jax
import jax.numpy as jnp
from jax.experimental import pallas as pl


def kernel(pos, feat, gamma, beta, W):
    raise NotImplementedError("write your pallas kernel here")


# speedup vs baseline: 4.4519x; 4.4519x over previous
import functools
import math

import jax
import jax.numpy as jnp
from jax.experimental import pallas as pl
from jax.experimental.pallas import tpu as pltpu

NUM_ITER = 10
POS_LAMBDA = 100.0

_HI = jax.lax.Precision.HIGHEST


def _cluster_kernel(pos_ref, feat_ref, posfull_ref, mf0_ref, mp0_ref,
                    gamma_ref, beta_ref, w_ref,
                    newpos_ref, newfeat_ref, newmask_ref,
                    *, k, n, c, d, num_iter, scale):
    feat = feat_ref[0]
    posb = pos_ref[0]
    x2f = jnp.sum(feat * feat, axis=1, keepdims=True)
    px = posb[:, 0:1]
    py = posb[:, 1:2]
    x2p = px * px + py * py
    iota_k = jax.lax.broadcasted_iota(jnp.int32, (n, k), 1).astype(jnp.float32)

    featb = feat.astype(jnp.bfloat16)
    pxb = px.astype(jnp.bfloat16).astype(jnp.float32)
    pyb = py.astype(jnp.bfloat16).astype(jnp.float32)

    def one_iter(_, carry):
        meansf, meansp, _ = carry
        m2f = jnp.sum(meansf * meansf, axis=1)[None, :]
        mx = meansp[:, 0][None, :]
        my = meansp[:, 1][None, :]
        m2p = mx * mx + my * my
        mxb = mx.astype(jnp.bfloat16).astype(jnp.float32)
        myb = my.astype(jnp.bfloat16).astype(jnp.float32)
        xf_m = jax.lax.dot_general(featb, meansf.astype(jnp.bfloat16),
                                   (((1,), (1,)), ((), ())),
                                   preferred_element_type=jnp.float32)
        dist = (x2f - 2.0 * xf_m + m2f
                + scale * (x2p - 2.0 * (pxb * mxb + pyb * myb) + m2p))
        rowmin = jnp.min(dist, axis=1, keepdims=True)
        cand = jnp.where(dist == rowmin, iota_k, float(k))
        assign = jnp.min(cand, axis=1, keepdims=True)
        onehot = (iota_k == assign).astype(jnp.float32)
        cnt = jnp.sum(onehot, axis=0)[:, None]
        denom = jnp.maximum(cnt, 1.0)
        sumf = jax.lax.dot_general(onehot, feat, (((0,), (0,)), ((), ())),
                                   precision=_HI)
        sump = jax.lax.dot_general(onehot, posb, (((0,), (0,)), ((), ())),
                                   precision=_HI)
        return sumf / denom, sump / denom, onehot

    init = (mf0_ref[0], mp0_ref[0], jnp.zeros((n, k), jnp.float32))
    _, _, onehot = jax.lax.fori_loop(0, num_iter, one_iter, init)

    pf = posfull_ref[...]
    posmax = jnp.max(jnp.max(pf, axis=1), axis=0)[None, :]
    posn = posb / posmax
    cnt = jnp.sum(onehot, axis=0)[:, None]
    safe = jnp.where(cnt > 0.0, cnt, 1.0)
    sumpn = jax.lax.dot_general(onehot, posn, (((0,), (0,)), ((), ())),
                                precision=_HI)
    mean_pos = sumpn / safe
    gathered = jax.lax.dot_general(onehot, mean_pos, (((1,), (0,)), ((), ())),
                                   precision=_HI)
    rel = posn - gathered
    relx = rel[:, 0:1]
    rely = rel[:, 1:2]
    cd = float(c + d)
    s1 = jnp.sum(feat, axis=1, keepdims=True) + relx + rely
    mu = s1 / cd
    df = feat - mu
    drx = relx - mu
    dry = rely - mu
    var = (jnp.sum(df * df, axis=1, keepdims=True) + drx * drx + dry * dry) / cd
    sstd = jnp.sqrt(var + 1e-5)
    g2 = gamma_ref[...]
    b2 = beta_ref[...]
    xf = df / sstd * g2[:, 0:c] + b2[:, 0:c]
    xpx = drx / sstd * g2[:, c:c + 1] + b2[:, c:c + 1]
    xpy = dry / sstd * g2[:, c + 1:c + 2] + b2[:, c + 1:c + 2]
    w = w_ref[...]
    wb = w.astype(jnp.bfloat16)
    y = jax.lax.dot_general(xf.astype(jnp.bfloat16), wb[0:c, :],
                            (((1,), (0,)), ((), ())),
                            preferred_element_type=jnp.float32)
    wpx = wb[c:c + 1, :].astype(jnp.float32)
    wpy = wb[c + 1:c + 2, :].astype(jnp.float32)
    xpxb = xpx.astype(jnp.bfloat16).astype(jnp.float32)
    xpyb = xpy.astype(jnp.bfloat16).astype(jnp.float32)
    y = y + xpxb * wpx + xpyb * wpy
    summed = jax.lax.dot_general(onehot, y, (((0,), (0,)), ((), ())),
                                 precision=_HI)
    merged = summed / safe
    valid = (cnt > 0.0).astype(jnp.float32)
    newfeat_ref[0] = merged * valid
    newpos_ref[0] = mean_pos * valid
    newmask_ref[0] = valid


def kernel(pos, feat, gamma, beta, W):
    b, n, c = feat.shape
    d = pos.shape[2]
    k = int(math.ceil(n / 4.0))
    init_idx = jnp.linspace(0, n - 1, k).astype(jnp.int32)
    mf0 = feat[:, init_idx, :]
    mp0 = pos[:, init_idx, :]
    g2 = gamma.reshape(1, c + d)
    b2 = beta.reshape(1, c + d)
    scale = POS_LAMBDA * float(c) / float(d)
    body = functools.partial(_cluster_kernel, k=k, n=n, c=c, d=d,
                             num_iter=NUM_ITER, scale=scale)
    out_shape = (
        jax.ShapeDtypeStruct((b, k, d), jnp.float32),
        jax.ShapeDtypeStruct((b, k, 2 * c), jnp.float32),
        jax.ShapeDtypeStruct((b, k, 1), jnp.float32),
    )
    return pl.pallas_call(
        body,
        grid=(b,),
        in_specs=[
            pl.BlockSpec((1, n, d), lambda i: (i, 0, 0)),
            pl.BlockSpec((1, n, c), lambda i: (i, 0, 0)),
            pl.BlockSpec((b, n, d), lambda i: (0, 0, 0)),
            pl.BlockSpec((1, k, c), lambda i: (i, 0, 0)),
            pl.BlockSpec((1, k, d), lambda i: (i, 0, 0)),
            pl.BlockSpec((1, c + d), lambda i: (0, 0)),
            pl.BlockSpec((1, c + d), lambda i: (0, 0)),
            pl.BlockSpec((c + d, 2 * c), lambda i: (0, 0)),
        ],
        out_specs=(
            pl.BlockSpec((1, k, d), lambda i: (i, 0, 0)),
            pl.BlockSpec((1, k, 2 * c), lambda i: (i, 0, 0)),
            pl.BlockSpec((1, k, 1), lambda i: (i, 0, 0)),
        ),
        out_shape=out_shape,
        compiler_params=pltpu.CompilerParams(
            dimension_semantics=("arbitrary",),
            vmem_limit_bytes=112 * 1024 * 1024,
        ),
    )(pos, feat, pos, mf0, mp0, g2, b2, W)
